# baseline (speedup 1.0000x reference)
import functools
import math

import jax
import jax.numpy as jnp
from jax.experimental import pallas as pl
from jax.experimental.pallas import tpu as pltpu

_LANE = 128


def _rup(a, m):
    return (a + m - 1) // m * m


def _pick_batch_tile(b, c, s, elem_bytes):
    for t in (8, 16, 32, 64):
        if b % t == 0 and t * c * s * elem_bytes >= 4 << 20:
            return t
    for t in (8, 16, 32, 64):
        if b % t == 0:
            return t
    return b


def _gem_head_body(s_ref, x_ref, w_ref, b_ref, o_ref, acc_ref, *, eps, log2_s):
    p_log2e = s_ref[0]
    inv_p = s_ref[1]

    y = jnp.log(jnp.maximum(x_ref[...], eps)) * p_log2e
    xp = jnp.exp2(y.astype(jnp.bfloat16)).astype(jnp.float32)
    acc_ref[...] = jnp.sum(xp, axis=-1)
    pooled = acc_ref[...]
    gem = jnp.exp2(inv_p * (jnp.log2(pooled) - log2_s))

    o_ref[...] = (jnp.dot(gem, w_ref[...], preferred_element_type=jnp.float32)
                  + b_ref[...])


def kernel(feat, p, gamma, beta, running_mean, running_var, cls_weight,
           *, gem_eps=1e-6, bn_eps=1e-5):
    b, c, h, w = feat.shape
    s = h * w
    n = cls_weight.shape[0]

    x = feat.reshape(b, c, s)

    scale = gamma.astype(jnp.float32) * jax.lax.rsqrt(
        running_var.astype(jnp.float32) + jnp.float32(bn_eps))
    shift = beta.astype(jnp.float32) - running_mean.astype(jnp.float32) * scale
    w_t = cls_weight.astype(jnp.float32).T
    w_fold = w_t * scale[:, None]
    bias = shift @ w_t

    n_pad = _rup(n, _LANE)
    if n_pad != n:
        w_fold = jnp.pad(w_fold, ((0, 0), (0, n_pad - n)))
        bias = jnp.pad(bias, (0, n_pad - n))
    bias = bias.reshape(1, n_pad)

    elem_bytes = jnp.dtype(feat.dtype).itemsize
    tb = _pick_batch_tile(b, c, s, elem_bytes)
    grid = (b // tb,)

    p32 = jnp.asarray(p, jnp.float32)
    scal = jnp.stack([p32 * jnp.float32(math.log2(math.e)), 1.0 / p32])

    out = pl.pallas_call(
        functools.partial(_gem_head_body, eps=float(gem_eps),
                          log2_s=math.log2(s)),
        out_shape=jax.ShapeDtypeStruct((b, n_pad), jnp.float32),
        grid=grid,
        in_specs=[
            pl.BlockSpec(memory_space=pltpu.MemorySpace.SMEM),
            pl.BlockSpec((tb, c, s), lambda bi: (bi, 0, 0)),
            pl.BlockSpec((c, n_pad), lambda bi: (0, 0)),
            pl.BlockSpec((1, n_pad), lambda bi: (0, 0)),
        ],
        out_specs=pl.BlockSpec((tb, n_pad), lambda bi: (bi, 0)),
        scratch_shapes=[pltpu.VMEM((tb, c), jnp.float32)],
        compiler_params=pltpu.CompilerParams(
            dimension_semantics=("parallel",),
            vmem_limit_bytes=56 << 20),
    )(scal, x, w_fold, bias)

    return out[:, :n] if n_pad != n else out
